# baseline (speedup 1.0000x reference)
import functools

import jax
import jax.numpy as jnp
from jax.experimental import pallas as pl
from jax.experimental.pallas import tpu as pltpu
from jax.experimental.pallas import tpu_sc as plsc


def _sc_gather(x_hbm, out_hbm, rows_v, sem, *, stride, row):
    cid = jax.lax.axis_index("c")
    sid = jax.lax.axis_index("s")
    wid = sid * 2 + cid

    @pl.when(wid == 0)
    def _():
        B = rows_v.shape[0]
        idx = jax.lax.iota(jnp.int32, B) * stride + row
        pltpu.async_copy(x_hbm.at[idx], rows_v, sem).wait()
        pltpu.sync_copy(rows_v, out_hbm)


def _mm_kernel(x_ref, aps_ref, w_ref, out_ref):
    logits = jax.lax.dot_general(
        x_ref[...],
        w_ref[...],
        dimension_numbers=(((1,), (1,)), ((), ())),
        preferred_element_type=jnp.float32,
    )
    mask = aps_ref[...] > 0
    out_ref[...] = jnp.where(mask, logits, jnp.zeros((), logits.dtype))


def kernel(input, agents_per_sample, W):
    B, A, S, D = input.shape
    V = W.shape[0]
    x2 = input.reshape(B * A * S, D)
    aps2 = agents_per_sample.reshape(B, 1)

    mesh = plsc.VectorSubcoreMesh(
        core_axis_name="c", subcore_axis_name="s", num_cores=2, num_subcores=16
    )
    gathered = pl.kernel(
        functools.partial(_sc_gather, stride=A * S, row=S - 1),
        out_type=jax.ShapeDtypeStruct((B, D), jnp.float32),
        mesh=mesh,
        scratch_types=[
            pltpu.VMEM((B, D), jnp.float32),
            pltpu.SemaphoreType.DMA,
        ],
    )(x2)

    return pl.pallas_call(
        _mm_kernel,
        out_shape=jax.ShapeDtypeStruct((B, V), input.dtype),
        grid=(1,),
        in_specs=[
            pl.BlockSpec((B, D), lambda i: (0, 0)),
            pl.BlockSpec((B, 1), lambda i: (0, 0)),
            pl.BlockSpec((V, D), lambda i: (0, 0)),
        ],
        out_specs=pl.BlockSpec((B, V), lambda i: (0, 0)),
    )(gathered, aps2, W)


# speedup vs baseline: 6.2897x; 6.2897x over previous
import functools

import jax
import jax.numpy as jnp
from jax.experimental import pallas as pl


def _emb_kernel(x_ref, aps_ref, w_ref, out_ref, *, row_off, nsteps):
    i = pl.program_id(0)
    B, R, Dblk = x_ref.shape
    V = w_ref.shape[0]
    xflat = x_ref[...].reshape(B * R, Dblk)
    logits_all = jax.lax.dot_general(
        xflat,
        w_ref[...],
        dimension_numbers=(((1,), (1,)), ((), ())),
        preferred_element_type=jnp.float32,
    ).reshape(B, R, V)
    rows = jax.lax.broadcasted_iota(jnp.int32, (B, R, V), 1)
    partial = jnp.sum(
        jnp.where(rows == row_off, logits_all, jnp.zeros((), logits_all.dtype)),
        axis=1,
    )

    @pl.when(i == 0)
    def _():
        out_ref[...] = partial

    @pl.when(i > 0)
    def _():
        out_ref[...] += partial

    @pl.when(i == nsteps - 1)
    def _():
        mask = aps_ref[...] > 0
        out_ref[...] = jnp.where(mask, out_ref[...], jnp.zeros((), out_ref.dtype))


def kernel(input, agents_per_sample, W):
    B, A, S, D = input.shape
    V = W.shape[0]
    x3 = input.reshape(B, A * S, D)
    blk = (S - 1) // 8
    row_off = (S - 1) % 8
    aps2 = agents_per_sample.reshape(B, 1)
    nsteps = 2
    dblk = D // nsteps

    return pl.pallas_call(
        functools.partial(_emb_kernel, row_off=row_off, nsteps=nsteps),
        out_shape=jax.ShapeDtypeStruct((B, V), input.dtype),
        grid=(nsteps,),
        in_specs=[
            pl.BlockSpec((B, 8, dblk), lambda i: (0, blk, i)),
            pl.BlockSpec((B, 1), lambda i: (0, 0)),
            pl.BlockSpec((V, dblk), lambda i: (0, i)),
        ],
        out_specs=pl.BlockSpec((B, V), lambda i: (0, 0)),
    )(x3, aps2, W)


# speedup vs baseline: 6.6693x; 1.0603x over previous
import functools

import jax
import jax.numpy as jnp
from jax.experimental import pallas as pl


def _emb_kernel(x_ref, aps_ref, w_ref, out_ref, *, row_off):
    B, R, D = x_ref.shape
    V = w_ref.shape[0]
    xflat = x_ref[...].reshape(B * R, D)
    logits_all = jax.lax.dot_general(
        xflat,
        w_ref[...],
        dimension_numbers=(((1,), (1,)), ((), ())),
        preferred_element_type=jnp.float32,
    ).reshape(B, R, V)
    rows = jax.lax.broadcasted_iota(jnp.int32, (B, R, V), 1)
    logits = jnp.sum(
        jnp.where(rows == row_off, logits_all, jnp.zeros((), logits_all.dtype)),
        axis=1,
    )
    mask = aps_ref[...] > 0
    out_ref[...] = jnp.where(mask, logits, jnp.zeros((), logits.dtype))


def kernel(input, agents_per_sample, W):
    B, A, S, D = input.shape
    V = W.shape[0]
    x3 = input.reshape(B, A * S, D)
    blk = (S - 1) // 8
    row_off = (S - 1) % 8
    aps2 = agents_per_sample.reshape(B, 1)

    return pl.pallas_call(
        functools.partial(_emb_kernel, row_off=row_off),
        out_shape=jax.ShapeDtypeStruct((B, V), input.dtype),
        grid=(1,),
        in_specs=[
            pl.BlockSpec((B, 8, D), lambda i: (0, blk, 0)),
            pl.BlockSpec((B, 1), lambda i: (0, 0)),
            pl.BlockSpec((V, D), lambda i: (0, 0)),
        ],
        out_specs=pl.BlockSpec((B, V), lambda i: (0, 0)),
    )(x3, aps2, W)
